# baseline (speedup 1.0000x reference)
import functools

import jax
import jax.numpy as jnp
from jax import lax
from jax.experimental import pallas as pl
from jax.experimental.pallas import tpu as pltpu
from jax.experimental.pallas import tpu_sc as plsc

N = 10000
E = 320000
L = 128
NC = 2
NS = 16
NW = NC * NS
CH = 128
NCH = 80
EW = NCH * CH
EP = EW * NW
HC = NCH // 2
HR = HC * CH
EH = EP // 2
RT = 640
NPAD = NS * RT

BE = 2048
BN = 2000

_f32 = jnp.float32


def _mesh():
    return plsc.VectorSubcoreMesh(
        core_axis_name="c", subcore_axis_name="s", num_cores=NC, num_subcores=NS
    )


def _sc_gather_body(xa_hbm, xb_hbm, src_hbm, dst_hbm, g_hbm,
                    ia, ib, r0, r1, s0, s1, p0, p1):
    cid = lax.axis_index("c")
    sid = lax.axis_index("s")
    wid = sid * NC + cid
    base = wid * HR

    pltpu.sync_copy(src_hbm.at[pl.ds(base, HR)], ia)
    pltpu.sync_copy(dst_hbm.at[pl.ds(base, HR)], ib)

    rs = (r0, r1)
    sg = (s0, s1)
    ss = (p0, p1)

    def idxa(c):
        return ia.at[pl.ds(c * CH, CH)]

    def idxb(c):
        return ib.at[pl.ds(c * CH, CH)]

    pltpu.async_copy(xa_hbm.at[idxa(0)], r0, s0)

    @pl.loop(0, HC, step=2)
    def _pair(j):
        for b in range(2):
            c = j + b
            nb = 1 - b

            @pl.when(c > 0)
            def _():
                pltpu.make_async_copy(
                    rs[nb], g_hbm.at[pl.ds(base, CH)], ss[nb]).wait()

            @pl.when(c + 1 < HC)
            def _():
                pltpu.async_copy(xa_hbm.at[idxa(c + 1)], rs[nb], sg[nb])

            pltpu.make_async_copy(xa_hbm.at[idxa(c)], rs[b], sg[b]).wait()
            pltpu.sync_copy(xb_hbm.at[idxb(c)], rs[b], add=True)
            off = base + c * CH
            pltpu.async_copy(rs[b], g_hbm.at[pl.ds(off, CH)], ss[b])

    pltpu.make_async_copy(r1, g_hbm.at[pl.ds(base, CH)], p1).wait()


def _sc_gather(xa, xb, src, dst):
    fn = pl.kernel(
        _sc_gather_body,
        out_type=jax.ShapeDtypeStruct((EH, L), _f32),
        mesh=_mesh(),
        scratch_types=[
            pltpu.VMEM((HR,), jnp.int32),
            pltpu.VMEM((HR,), jnp.int32),
            pltpu.VMEM((CH, L), _f32),
            pltpu.VMEM((CH, L), _f32),
        ] + [pltpu.SemaphoreType.DMA] * 4,
    )
    return fn(xa, xb, src, dst)


def _sc_scatter_body(vals_hbm, idx_hbm, z128_hbm, out0_hbm, out1_hbm,
                     idx_full, v0, v1, sv0, sv1, acc):
    cid = lax.axis_index("c")
    sid = lax.axis_index("s")
    wid = sid * NC + cid
    base = wid * HR
    rbase = sid * RT

    pltpu.sync_copy(idx_hbm.at[pl.ds(base, HR)], idx_full)

    pltpu.sync_copy(z128_hbm, v0)

    @pl.loop(0, RT // CH)
    def _z(k):
        pltpu.sync_copy(v0, acc.at[pl.ds(rbase + k * CH, CH)])

    pltpu.async_copy(vals_hbm.at[pl.ds(base, CH)], v0, sv0)
    plsc.subcore_barrier()

    vs = (v0, v1)
    svs = (sv0, sv1)

    @pl.loop(0, HC, step=2)
    def _pair(j):
        for b in range(2):
            c = j + b
            nb = 1 - b

            @pl.when(c + 1 < HC)
            def _():
                pltpu.async_copy(
                    vals_hbm.at[pl.ds(base + (c + 1) * CH, CH)],
                    vs[nb], svs[nb])

            pltpu.make_async_copy(
                vals_hbm.at[pl.ds(base, CH)], vs[b], svs[b]).wait()
            pltpu.sync_copy(
                vs[b], acc.at[idx_full.at[pl.ds(c * CH, CH)]], add=True)

    plsc.subcore_barrier()

    @pl.loop(0, RT // CH)
    def _w(k):
        sl = pl.ds(rbase + k * CH, CH)
        pltpu.sync_copy(acc.at[sl], v0)

        @pl.when(cid == 0)
        def _():
            pltpu.sync_copy(v0, out0_hbm.at[sl])

        @pl.when(cid == 1)
        def _():
            pltpu.sync_copy(v0, out1_hbm.at[sl])


def _sc_scatter(vals, idx, z128):
    fn = pl.kernel(
        _sc_scatter_body,
        out_type=(jax.ShapeDtypeStruct((NPAD, L), _f32),
                  jax.ShapeDtypeStruct((NPAD, L), _f32)),
        mesh=_mesh(),
        scratch_types=[
            pltpu.VMEM((HR,), jnp.int32),
            pltpu.VMEM((CH, L), _f32),
            pltpu.VMEM((CH, L), _f32),
            pltpu.SemaphoreType.DMA,
            pltpu.SemaphoreType.DMA,
            pltpu.VMEM_SHARED((NPAD, L), _f32),
        ],
    )
    return fn(vals, idx, z128)


def _sc_count_body(idx_hbm, ones_hbm, z128_hbm, out0_hbm, out1_hbm,
                   idx_full, rows_v, acc):
    cid = lax.axis_index("c")
    sid = lax.axis_index("s")
    wid = sid * NC + cid
    base = wid * EW
    rbase = sid * RT

    pltpu.sync_copy(idx_hbm.at[pl.ds(base, EW)], idx_full)
    pltpu.sync_copy(z128_hbm, rows_v)

    @pl.loop(0, RT // CH)
    def _z(k):
        pltpu.sync_copy(rows_v, acc.at[pl.ds(rbase + k * CH, CH)])

    pltpu.sync_copy(ones_hbm, rows_v)
    plsc.subcore_barrier()

    @pl.loop(0, NCH)
    def _chunk(j):
        pltpu.sync_copy(
            rows_v, acc.at[idx_full.at[pl.ds(j * CH, CH)]], add=True)

    plsc.subcore_barrier()

    @pl.loop(0, RT // CH)
    def _w(k):
        sl = pl.ds(rbase + k * CH, CH)
        pltpu.sync_copy(acc.at[sl], rows_v)

        @pl.when(cid == 0)
        def _():
            pltpu.sync_copy(rows_v, out0_hbm.at[sl])

        @pl.when(cid == 1)
        def _():
            pltpu.sync_copy(rows_v, out1_hbm.at[sl])

        pltpu.sync_copy(ones_hbm, rows_v)


def _sc_count(idx, ones128, z128):
    fn = pl.kernel(
        _sc_count_body,
        out_type=(jax.ShapeDtypeStruct((NPAD, L), _f32),
                  jax.ShapeDtypeStruct((NPAD, L), _f32)),
        mesh=_mesh(),
        scratch_types=[
            pltpu.VMEM((EW,), jnp.int32),
            pltpu.VMEM((CH, L), _f32),
            pltpu.VMEM_SHARED((NPAD, L), _f32),
        ],
    )
    return fn(idx, ones128, z128)


def _ln(t, g, b):
    m = jnp.mean(t, axis=-1, keepdims=True)
    v = jnp.mean((t - m) * (t - m), axis=-1, keepdims=True)
    return (t - m) * lax.rsqrt(v + 1e-5) * g + b


def _dot(a, b):
    return jnp.dot(a, b, preferred_element_type=_f32)


def _edge_enc_body(ef, w1, b1, w2, b2, w3, b3, lg, lb, out):
    t = _dot(ef[...], w1[...]) + b1[...]
    t = _dot(t, w2[...]) + b2[...]
    t = _dot(t, w3[...]) + b3[...]
    out[...] = _ln(t, lg[...], lb[...])


def _edge_step_body(e_ref, g_ref, w1c, w2, b2, w3, b3, lg, lb,
                    enew_ref, enext_ref):
    e = e_ref[...]
    h = g_ref[...] + _dot(e, w1c[...])
    h = _dot(h, w2[...]) + b2[...]
    t = _dot(h, w3[...]) + b3[...]
    en = _ln(t, lg[...], lb[...])
    enew_ref[...] = en
    enext_ref[...] = e + en


def _node_first_body(nf, w1, b1, w2, b2, w3, b3, lg, lb,
                     sA0, sA1, sB0, sB1, c0, c1, w1a, w1b, be1,
                     x_ref, xa_ref, xb_ref):
    t = _dot(nf[...], w1[...]) + b1[...]
    t = _dot(t, w2[...]) + b2[...]
    t = _dot(t, w3[...]) + b3[...]
    xe = _ln(t, lg[...], lb[...])
    cnt = jnp.maximum(c0[:, 0:1] + c1[:, 0:1], 1.0)
    x = xe + (sA0[...] + sA1[...] + sB0[...] + sB1[...]) / cnt
    x_ref[...] = x
    xa_ref[...] = _dot(x, w1a[...]) + be1[...]
    xb_ref[...] = _dot(x, w1b[...])


def _node_step_body(x_ref, qA0, qA1, qB0, qB1, d0, d1,
                    v1a, v1b, cb1, v2, cb2, v3, cb3, lg, lb,
                    w1a, w1b, be1,
                    xn_ref, xa_ref, xb_ref):
    x = x_ref[...]
    cnt = jnp.maximum(d0[:, 0:1] + d1[:, 0:1], 1.0)
    agg = (qA0[...] + qA1[...] + qB0[...] + qB1[...]) / cnt
    u = _dot(x, v1a[...]) + _dot(agg, v1b[...]) + cb1[...]
    u = _dot(u, v2[...]) + cb2[...]
    u = _dot(u, v3[...]) + cb3[...]
    xn = x + _ln(u, lg[...], lb[...])
    xn_ref[...] = xn
    xa_ref[...] = _dot(xn, w1a[...]) + be1[...]
    xb_ref[...] = _dot(xn, w1b[...])


def _node_last_body(x_ref, qA0, qA1, qB0, qB1, d0, d1,
                    v1a, v1b, cb1, v2, cb2, v3, cb3, lg, lb,
                    dw1, db1, dw2, db2, dw3, db3,
                    out_ref):
    x = x_ref[...]
    cnt = jnp.maximum(d0[:, 0:1] + d1[:, 0:1], 1.0)
    agg = (qA0[...] + qA1[...] + qB0[...] + qB1[...]) / cnt
    u = _dot(x, v1a[...]) + _dot(agg, v1b[...]) + cb1[...]
    u = _dot(u, v2[...]) + cb2[...]
    u = _dot(u, v3[...]) + cb3[...]
    xn = x + _ln(u, lg[...], lb[...])
    t = _dot(xn, dw1[...]) + db1[...]
    t = _dot(t, dw2[...]) + db2[...]
    out_ref[...] = _dot(t, dw3[...]) + db3[...]


def _full(shape):
    return pl.BlockSpec(shape, lambda i: (0,) * len(shape))


def _rows(nrow, ncol):
    return pl.BlockSpec((nrow, ncol), lambda i: (i, 0))


def _call_edge_enc(ef, w):
    grid = (EH // BE,)
    return pl.pallas_call(
        _edge_enc_body,
        grid=grid,
        in_specs=[_rows(BE, 16)] + [_full(a.shape) for a in w],
        out_specs=_rows(BE, L),
        out_shape=jax.ShapeDtypeStruct((EH, L), _f32),
    )(ef, *w)


def _call_edge_step(e, g, w):
    grid = (EH // BE,)
    return pl.pallas_call(
        _edge_step_body,
        grid=grid,
        in_specs=[_rows(BE, L), _rows(BE, L)]
        + [_full(a.shape) for a in w],
        out_specs=(_rows(BE, L), _rows(BE, L)),
        out_shape=(jax.ShapeDtypeStruct((EH, L), _f32),
                   jax.ShapeDtypeStruct((EH, L), _f32)),
    )(e, g, *w)


def _call_node_first(nf, encw, s, c, projw):
    grid = (N // BN,)
    specs = ([_rows(BN, L)] + [_full(a.shape) for a in encw]
             + [_rows(BN, L)] * 6
             + [_full(a.shape) for a in projw])
    return pl.pallas_call(
        _node_first_body,
        grid=grid,
        in_specs=specs,
        out_specs=(_rows(BN, L), _rows(BN, L), _rows(BN, L)),
        out_shape=(jax.ShapeDtypeStruct((N, L), _f32),) * 3,
    )(nf, *encw, s[0], s[1], s[2], s[3], c[0], c[1], *projw)


def _call_node_step(x, q, d, nodew, projw):
    grid = (N // BN,)
    specs = ([_rows(BN, L)] * 7
             + [_full(a.shape) for a in nodew]
             + [_full(a.shape) for a in projw])
    return pl.pallas_call(
        _node_step_body,
        grid=grid,
        in_specs=specs,
        out_specs=(_rows(BN, L), _rows(BN, L), _rows(BN, L)),
        out_shape=(jax.ShapeDtypeStruct((N, L), _f32),) * 3,
    )(x, q[0], q[1], q[2], q[3], d[0], d[1], *nodew, *projw)


def _call_node_last(x, q, d, nodew, decw):
    grid = (N // BN,)
    specs = ([_rows(BN, L)] * 7
             + [_full(a.shape) for a in nodew]
             + [_full(a.shape) for a in decw])
    return pl.pallas_call(
        _node_last_body,
        grid=grid,
        in_specs=specs,
        out_specs=_rows(BN, 3),
        out_shape=jax.ShapeDtypeStruct((N, 3), _f32),
    )(x, q[0], q[1], q[2], q[3], d[0], d[1], *nodew, *decw)


def _row(v):
    return v.reshape(1, -1)


def _block_weights(blk):
    (w1, b1), (w2, b2), (w3, b3) = blk["mlp"]
    lg, lb = blk["ln"]
    return [w1, _row(b1), w2, _row(b2), w3, _row(b3), _row(lg), _row(lb)]


def _split_halves_1d(a, pad_val):
    p = jnp.pad(a, (0, EP - E), constant_values=pad_val)
    p = p.reshape(NW, 2, HR)
    return p[:, 0].reshape(-1), p[:, 1].reshape(-1)


def kernel(node_feat, edge_feat, edge_index, params):
    src = edge_index[0].astype(jnp.int32)
    dst = edge_index[1].astype(jnp.int32)

    ef_p = jnp.pad(edge_feat, ((0, EP - E), (0, 0)))
    ef3 = ef_p.reshape(NW, 2, HR, edge_feat.shape[1])
    efA = ef3[:, 0].reshape(EH, -1)
    efB = ef3[:, 1].reshape(EH, -1)
    srcA_g, srcB_g = _split_halves_1d(src, 0)
    dstA_g, dstB_g = _split_halves_1d(dst, 0)
    srcA_s, srcB_s = _split_halves_1d(src, N)
    dstA_s, dstB_s = _split_halves_1d(dst, N)
    src_cnt = jnp.concatenate([srcA_s, srcB_s])
    dst_cnt = jnp.concatenate([dstA_s, dstB_s])

    z128 = jnp.zeros((CH, L), _f32)
    ones128 = jnp.ones((CH, L), _f32)

    encw_e = _block_weights(params["edge_enc"])
    encw_n = _block_weights(params["node_enc"])

    edge_w, node_w, proj_w = [], [], []
    for p in params["proc"]:
        (w1, b1), (w2, b2), (w3, b3) = p["edge"]["mlp"]
        lg, lb = p["edge"]["ln"]
        proj_w.append([w1[:L], w1[L:2 * L], _row(b1)])
        edge_w.append([w1[2 * L:], w2, _row(b2), w3, _row(b3), _row(lg), _row(lb)])
        (v1, c1), (v2, c2), (v3, c3) = p["node"]["mlp"]
        ng, nb = p["node"]["ln"]
        node_w.append([v1[:L], v1[L:], _row(c1), v2, _row(c2), v3, _row(c3),
                       _row(ng), _row(nb)])
    (dw1, db1), (dw2, db2), (dw3, db3) = params["dec"]
    decw = [dw1, _row(db1), dw2, _row(db2), dw3, _row(db3)]

    eA = _call_edge_enc(efA, encw_e)
    sA0, sA1 = _sc_scatter(eA, srcA_s, z128)
    eB = _call_edge_enc(efB, encw_e)
    sB0, sB1 = _sc_scatter(eB, srcB_s, z128)
    c0, c1 = _sc_count(src_cnt, ones128, z128)
    d0, d1 = _sc_count(dst_cnt, ones128, z128)
    s, c, d = (sA0, sA1, sB0, sB1), (c0, c1), (d0, d1)
    x, xa, xb = _call_node_first(node_feat, encw_n, s, c, proj_w[0])

    for i in range(4):
        gA = _sc_gather(xa, xb, srcA_g, dstA_g)
        gB = _sc_gather(xa, xb, srcB_g, dstB_g)
        eA_new, eA = _call_edge_step(eA, gA, edge_w[i])
        qA0, qA1 = _sc_scatter(eA_new, dstA_s, z128)
        eB_new, eB = _call_edge_step(eB, gB, edge_w[i])
        qB0, qB1 = _sc_scatter(eB_new, dstB_s, z128)
        q = (qA0, qA1, qB0, qB1)
        if i < 3:
            x, xa, xb = _call_node_step(x, q, d, node_w[i], proj_w[i + 1])
        else:
            out = _call_node_last(x, q, d, node_w[i], decw)
    return out
